# Initial kernel scaffold; baseline (speedup 1.0000x reference)
import jax
import jax.numpy as jnp
from jax.experimental import pallas as pl


def kernel(input, h0, adj_rows, adj_cols, adj_vals, d_rows, d_cols, d_vals, lamda, alpha, l, gamma, weight):
    raise NotImplementedError("write your pallas kernel here")


# speedup vs baseline: 8.9757x; 8.9757x over previous
import functools

import jax
import jax.numpy as jnp
from jax import lax
from jax.experimental import pallas as pl
from jax.experimental.pallas import tpu as pltpu
from jax.experimental.pallas import tpu_sc as plsc

N = 4096
DF = 128
NNZ_ADJ = 131072
NNZ_D = 16384
NC = 2
NS = 16
NW = NC * NS
L = 16
G = 128
RPT = N // NS

_mesh = plsc.VectorSubcoreMesh(
    core_axis_name="c", subcore_axis_name="s", num_cores=NC, num_subcores=NS
)


def _zero_rows(buf, nrows):
    zeros = jnp.zeros((L,), jnp.float32)

    def body(i, _):
        for j in range(DF // L):
            buf[i, pl.ds(j * L, L)] = zeros
        return 0

    lax.fori_loop(0, nrows, body, 0)


def _load_idx(src_hbm, off, dst, ngroups):
    for i in range(ngroups):
        pltpu.sync_copy(src_hbm.at[pl.ds(off + i * G, G)], dst.at[i])


def _make_spmm(nnz, dual, fold_rnd_out, chunk):
    ng = chunk // G
    per_tile = nnz // NW
    nchunks = per_tile // chunk
    assert per_tile % chunk == 0

    def body(rows_hbm, cols_hbm, vals_hbm, *rest):
        if fold_rnd_out:
            rnd_hbm = rest[0]
            rest = rest[1:]
        tables = rest[: (2 if dual else 1)]
        rest = rest[(2 if dual else 1):]
        out_hbm = rest[0]
        scratch = rest[1:]
        if fold_rnd_out:
            rndbuf = scratch[0]
            scratch = scratch[1:]
        acc, colbuf, rowbuf, valbuf = scratch[:4]
        gbufs = scratch[4: 4 + (2 if dual else 1)]
        sem = scratch[-1]

        c = lax.axis_index("c")
        s = lax.axis_index("s")
        wid = s * NC + c

        _zero_rows(gbufs[0], RPT)
        pltpu.sync_copy(gbufs[0].at[pl.ds(0, RPT)], acc.at[pl.ds(s * RPT, RPT)])
        plsc.subcore_barrier()

        def one_chunk(ci, _):
            off = pl.multiple_of(wid * per_tile + ci * chunk, 8)
            _load_idx(cols_hbm, off, colbuf, ng)
            _load_idx(rows_hbm, off, rowbuf, ng)
            pltpu.sync_copy(vals_hbm.at[pl.ds(off, chunk)], valbuf)

            descs = []
            for i in range(ng):
                gsl = pl.ds(i * G, G)
                for t, gb in zip(tables, gbufs):
                    descs.append(pltpu.async_copy(t.at[colbuf.at[i]], gb.at[gsl], sem))
            for d in descs:
                d.wait()

            def scale(k16, _):
                vv = valbuf[pl.ds(k16 * L, L)]
                for j in range(L):
                    v = vv[j]
                    row = k16 * L + j
                    for q in range(DF // L):
                        sl = pl.ds(q * L, L)
                        if dual:
                            gbufs[0][row, sl] = (gbufs[0][row, sl] + gbufs[1][row, sl]) * v
                        else:
                            gbufs[0][row, sl] = gbufs[0][row, sl] * v
                return 0

            lax.fori_loop(0, chunk // L, scale, 0)
            for i in range(ng):
                pltpu.sync_copy(gbufs[0].at[pl.ds(i * G, G)], acc.at[rowbuf.at[i]], add=True)
            return 0

        lax.fori_loop(0, nchunks, one_chunk, 0)
        plsc.subcore_barrier()
        sl = pl.ds(s * RPT, RPT)
        if fold_rnd_out:
            pltpu.sync_copy(rnd_hbm.at[sl], rndbuf)
            pltpu.sync_copy(acc.at[sl], gbufs[0].at[pl.ds(0, RPT)])

            def rscale(k16, _):
                vv = rndbuf[pl.ds(k16 * L, L)]
                for j in range(L):
                    v = vv[j]
                    row = k16 * L + j
                    for q in range(DF // L):
                        qsl = pl.ds(q * L, L)
                        gbufs[0][row, qsl] = gbufs[0][row, qsl] * v
                return 0

            lax.fori_loop(0, RPT // L, rscale, 0)
            pltpu.sync_copy(gbufs[0].at[pl.ds(0, RPT)], out_hbm.at[c, sl])
        else:
            pltpu.sync_copy(acc.at[sl], out_hbm.at[c, sl])

    scratch_types = []
    if fold_rnd_out:
        scratch_types.append(pltpu.VMEM((RPT,), jnp.float32))
    scratch_types += [
        pltpu.VMEM_SHARED((N, DF), jnp.float32),
        pltpu.VMEM((ng, G), jnp.int32),
        pltpu.VMEM((ng, G), jnp.int32),
        pltpu.VMEM((chunk,), jnp.float32),
    ]
    scratch_types += [pltpu.VMEM((chunk, DF), jnp.float32)] * (2 if dual else 1)
    scratch_types.append(pltpu.SemaphoreType.DMA)

    return pl.kernel(
        body,
        out_type=jax.ShapeDtypeStruct((NC, N, DF), jnp.float32),
        mesh=_mesh,
        scratch_types=scratch_types,
    )


_spmm_d1 = _make_spmm(NNZ_D, dual=False, fold_rnd_out=True, chunk=512)
_spmm_adj = _make_spmm(NNZ_ADJ, dual=False, fold_rnd_out=False, chunk=512)
_spmm_d1_dual = _make_spmm(NNZ_D, dual=True, fold_rnd_out=False, chunk=256)

_BLK = 512


def _combine_body(coef_ref, s2_ref, a_ref, h0_ref, w_ref, out_ref):
    th = coef_ref[0, 0]
    c1 = coef_ref[0, 1]
    c2 = coef_ref[0, 2]
    c3 = coef_ref[0, 3]
    sup = (c1 * (s2_ref[0] + s2_ref[1])
           + c2 * (a_ref[0] + a_ref[1])
           + c3 * h0_ref[...])
    out_ref[...] = th * jnp.dot(
        sup, w_ref[...], preferred_element_type=jnp.float32
    ) + (1.0 - th) * sup


def _combine(coefs, s2p, ap, h0, w):
    return pl.pallas_call(
        _combine_body,
        grid=(N // _BLK,),
        in_specs=[
            pl.BlockSpec(memory_space=pltpu.MemorySpace.SMEM),
            pl.BlockSpec((NC, _BLK, DF), lambda i: (0, i, 0)),
            pl.BlockSpec((NC, _BLK, DF), lambda i: (0, i, 0)),
            pl.BlockSpec((_BLK, DF), lambda i: (i, 0)),
            pl.BlockSpec((DF, DF), lambda i: (0, 0)),
        ],
        out_specs=pl.BlockSpec((_BLK, DF), lambda i: (i, 0)),
        out_shape=jax.ShapeDtypeStruct((N, DF), jnp.float32),
    )(coefs, s2p, ap, h0, w)


def kernel(input, h0, adj_rows, adj_cols, adj_vals, d_rows, d_cols, d_vals,
           lamda, alpha, l, gamma, weight):
    x = input
    d1r, d1c, d1v = d_rows[1], d_cols[1], d_vals[1]
    rnd1 = jax.random.uniform(jax.random.key(42), (2 * N, 1), dtype=jnp.float32)[N:, 0]

    tp = _spmm_d1(d1r, d1c, d1v, rnd1, x)
    ap = _spmm_adj(adj_rows, adj_cols, adj_vals, x)
    s2p = _spmm_d1_dual(d1r, d1c, d1v, tp[0], tp[1])

    theta = jnp.log(lamda / l + 1.0)
    af = jnp.float32(alpha)
    gf = jnp.float32(gamma)
    coefs = jnp.stack(
        [jnp.float32(theta), (1.0 - af) * gf, (1.0 - af) * (1.0 - gf), af]
    ).reshape(1, 4)

    return _combine(coefs, s2p, ap, h0, weight)


# speedup vs baseline: 10.9630x; 1.2214x over previous
import functools

import jax
import jax.numpy as jnp
from jax import lax
from jax.experimental import pallas as pl
from jax.experimental.pallas import tpu as pltpu
from jax.experimental.pallas import tpu_sc as plsc

N = 4096
DF = 128
NNZ_ADJ = 131072
NNZ_D = 16384
NC = 2
NS = 16
NW = NC * NS
L = 16
G = 128
RPT = N // NS

_mesh = plsc.VectorSubcoreMesh(
    core_axis_name="c", subcore_axis_name="s", num_cores=NC, num_subcores=NS
)


def _zero_rows(buf, nrows):
    zeros = jnp.zeros((L,), jnp.float32)

    def body(i, _):
        for j in range(DF // L):
            buf[i, pl.ds(j * L, L)] = zeros
        return 0

    lax.fori_loop(0, nrows, body, 0)


def _load_idx(src_hbm, off, dst, ngroups):
    for i in range(ngroups):
        pltpu.sync_copy(src_hbm.at[pl.ds(off + i * G, G)], dst.at[i])


def _make_spmm(nnz, dual, fold_rnd_out, chunk):
    ng = chunk // G
    per_tile = nnz // NW
    ngt = per_tile // G
    nchunks = per_tile // chunk
    ntab = 2 if dual else 1
    P = 2
    assert per_tile % chunk == 0

    def body(rows_hbm, cols_hbm, vals_hbm, *rest):
        if fold_rnd_out:
            rnd_hbm = rest[0]
            rest = rest[1:]
        tables = rest[:ntab]
        rest = rest[ntab:]
        out_hbm = rest[0]
        scratch = list(rest[1:])
        if fold_rnd_out:
            rndbuf = scratch.pop(0)
        acc = scratch.pop(0)
        colbuf = scratch.pop(0)
        rowbuf = scratch.pop(0)
        valbuf = scratch.pop(0)
        gbufs = [[scratch.pop(0) for _ in range(ntab)] for _ in range(P)]
        gsems = [scratch.pop(0) for _ in range(P)]
        ssem = scratch.pop(0)

        c = lax.axis_index("c")
        s = lax.axis_index("s")
        wid = s * NC + c

        zr = min(chunk, RPT)
        _zero_rows(gbufs[0][0], zr)
        for z in range(0, RPT, zr):
            pltpu.sync_copy(gbufs[0][0].at[pl.ds(0, zr)], acc.at[pl.ds(s * RPT + z, zr)])

        goff = pl.multiple_of(wid * ngt, 1)
        pltpu.sync_copy(cols_hbm.at[pl.ds(goff, ngt)], colbuf)
        pltpu.sync_copy(rows_hbm.at[pl.ds(goff, ngt)], rowbuf)
        pltpu.sync_copy(vals_hbm.at[pl.ds(pl.multiple_of(wid * per_tile, 8), per_tile)],
                        valbuf)
        plsc.subcore_barrier()

        def fire_gathers(ci):
            p = ci % P
            descs = []
            for i in range(ng):
                g = ci * ng + i
                gsl = pl.ds(i * G, G)
                for t, gb in zip(tables, gbufs[p]):
                    descs.append(pltpu.async_copy(t.at[colbuf.at[g]], gb.at[gsl], gsems[p]))
            return descs

        gd = {0: fire_gathers(0)}
        sd = {}
        for ci in range(nchunks):
            p = ci % P
            if ci + 1 < nchunks:
                if ci - 1 >= 0:
                    for d in sd[ci - 1]:
                        d.wait()
                gd[ci + 1] = fire_gathers(ci + 1)
            for d in gd[ci]:
                d.wait()

            def scale(k16, _, ci=ci, p=p):
                vv = valbuf[pl.ds(ci * chunk + k16 * L, L)]
                for j in range(L):
                    v = vv[j]
                    row = k16 * L + j
                    for q in range(DF // L):
                        sl = pl.ds(q * L, L)
                        if dual:
                            gbufs[p][0][row, sl] = (gbufs[p][0][row, sl]
                                                    + gbufs[p][1][row, sl]) * v
                        else:
                            gbufs[p][0][row, sl] = gbufs[p][0][row, sl] * v
                return 0

            lax.fori_loop(0, chunk // L, scale, 0)
            sd[ci] = []
            for i in range(ng):
                g = ci * ng + i
                sd[ci].append(pltpu.async_copy(
                    gbufs[p][0].at[pl.ds(i * G, G)], acc.at[rowbuf.at[g]], ssem, add=True))
        for ci in range(max(0, nchunks - 2), nchunks):
            for d in sd[ci]:
                d.wait()
        plsc.subcore_barrier()
        sl = pl.ds(s * RPT, RPT)
        if fold_rnd_out:
            pltpu.sync_copy(rnd_hbm.at[sl], rndbuf)
            pltpu.sync_copy(acc.at[sl], gbufs[0][0].at[pl.ds(0, RPT)])

            def rscale(k16, _):
                vv = rndbuf[pl.ds(k16 * L, L)]
                for j in range(L):
                    v = vv[j]
                    row = k16 * L + j
                    for q in range(DF // L):
                        qsl = pl.ds(q * L, L)
                        gbufs[0][0][row, qsl] = gbufs[0][0][row, qsl] * v
                return 0

            lax.fori_loop(0, RPT // L, rscale, 0)
            pltpu.sync_copy(gbufs[0][0].at[pl.ds(0, RPT)], out_hbm.at[c, sl])
        else:
            pltpu.sync_copy(acc.at[sl], out_hbm.at[c, sl])

    scratch_types = []
    if fold_rnd_out:
        scratch_types.append(pltpu.VMEM((RPT,), jnp.float32))
    scratch_types += [
        pltpu.VMEM_SHARED((N, DF), jnp.float32),
        pltpu.VMEM((ngt, G), jnp.int32),
        pltpu.VMEM((ngt, G), jnp.int32),
        pltpu.VMEM((per_tile,), jnp.float32),
    ]
    scratch_types += [pltpu.VMEM((chunk, DF), jnp.float32)] * (P * ntab)
    scratch_types += [pltpu.SemaphoreType.DMA] * P
    scratch_types.append(pltpu.SemaphoreType.DMA)

    return pl.kernel(
        body,
        out_type=jax.ShapeDtypeStruct((NC, N, DF), jnp.float32),
        mesh=_mesh,
        scratch_types=scratch_types,
    )


_spmm_d1 = _make_spmm(NNZ_D, dual=False, fold_rnd_out=True, chunk=256)
_spmm_adj = _make_spmm(NNZ_ADJ, dual=False, fold_rnd_out=False, chunk=256)
_spmm_d1_dual = _make_spmm(NNZ_D, dual=True, fold_rnd_out=False, chunk=128)

_BLK = 512


def _combine_body(coef_ref, s2_ref, a_ref, h0_ref, w_ref, out_ref):
    th = coef_ref[0, 0]
    c1 = coef_ref[0, 1]
    c2 = coef_ref[0, 2]
    c3 = coef_ref[0, 3]
    sup = (c1 * (s2_ref[0] + s2_ref[1])
           + c2 * (a_ref[0] + a_ref[1])
           + c3 * h0_ref[...])
    out_ref[...] = th * jnp.dot(
        sup, w_ref[...], preferred_element_type=jnp.float32
    ) + (1.0 - th) * sup


def _combine(coefs, s2p, ap, h0, w):
    return pl.pallas_call(
        _combine_body,
        grid=(N // _BLK,),
        in_specs=[
            pl.BlockSpec(memory_space=pltpu.MemorySpace.SMEM),
            pl.BlockSpec((NC, _BLK, DF), lambda i: (0, i, 0)),
            pl.BlockSpec((NC, _BLK, DF), lambda i: (0, i, 0)),
            pl.BlockSpec((_BLK, DF), lambda i: (i, 0)),
            pl.BlockSpec((DF, DF), lambda i: (0, 0)),
        ],
        out_specs=pl.BlockSpec((_BLK, DF), lambda i: (i, 0)),
        out_shape=jax.ShapeDtypeStruct((N, DF), jnp.float32),
    )(coefs, s2p, ap, h0, w)


def kernel(input, h0, adj_rows, adj_cols, adj_vals, d_rows, d_cols, d_vals,
           lamda, alpha, l, gamma, weight):
    x = input
    d1r, d1c, d1v = d_rows[1], d_cols[1], d_vals[1]
    rnd1 = jax.random.uniform(jax.random.key(42), (2 * N, 1), dtype=jnp.float32)[N:, 0]

    d1r2 = d1r.reshape(-1, G)
    d1c2 = d1c.reshape(-1, G)
    tp = _spmm_d1(d1r2, d1c2, d1v, rnd1, x)
    ap = _spmm_adj(adj_rows.reshape(-1, G), adj_cols.reshape(-1, G), adj_vals, x)
    s2p = _spmm_d1_dual(d1r2, d1c2, d1v, tp[0], tp[1])

    theta = jnp.log(lamda / l + 1.0)
    af = jnp.float32(alpha)
    gf = jnp.float32(gamma)
    coefs = jnp.stack(
        [jnp.float32(theta), (1.0 - af) * gf, (1.0 - af) * (1.0 - gf), af]
    ).reshape(1, 4)

    return _combine(coefs, s2p, ap, h0, weight)


# speedup vs baseline: 12.5229x; 1.1423x over previous
import jax
import jax.numpy as jnp
from jax import lax
from jax.experimental import pallas as pl
from jax.experimental.pallas import tpu as pltpu
from jax.experimental.pallas import tpu_sc as plsc

N = 4096
DF = 128
NNZ_ADJ = 131072
NNZ_D = 16384
NC = 2
NS = 16
NW = NC * NS
L = 16
G = 128
RPT = N // NS

CHUNK = 256
NGC = CHUNK // G
PTD = NNZ_D // NS
PTA = NNZ_ADJ // NW
DGT = PTD // G
AGT = PTA // G

_mesh = plsc.VectorSubcoreMesh(
    core_axis_name="c", subcore_axis_name="s", num_cores=NC, num_subcores=NS
)


def _zero_rows(buf, nrows):
    zeros = jnp.zeros((L,), jnp.float32)

    def body(i, _):
        for j in range(DF // L):
            buf[i, pl.ds(j * L, L)] = zeros
        return 0

    lax.fori_loop(0, nrows, body, 0)


def _mega_body(d1r, d1c, d1v, a_r, a_c, a_v, rnd, x,
               t_stage, s2_out, ap_out,
               acc, dcol, drow, dval, acol, arow, aval, rndbuf,
               gb0, gb1, gsem0, gsem1, ssem):
    c = lax.axis_index("c")
    s = lax.axis_index("s")
    wid = s * NC + c
    gbufs = (gb0, gb1)
    gsems = (gsem0, gsem1)

    _zero_rows(gb0, RPT)
    pltpu.sync_copy(gb0.at[pl.ds(0, RPT)], acc.at[pl.ds(s * RPT, RPT)])
    pltpu.sync_copy(d1c.at[pl.ds(s * DGT, DGT)], dcol)
    pltpu.sync_copy(d1r.at[pl.ds(s * DGT, DGT)], drow)
    pltpu.sync_copy(d1v.at[pl.ds(pl.multiple_of(s * PTD, 8), PTD)], dval)
    pltpu.sync_copy(a_c.at[pl.ds(wid * AGT, AGT)], acol)
    pltpu.sync_copy(a_r.at[pl.ds(wid * AGT, AGT)], arow)
    pltpu.sync_copy(a_v.at[pl.ds(pl.multiple_of(wid * PTA, 8), PTA)], aval)
    pltpu.sync_copy(rnd.at[pl.ds(pl.multiple_of(s * RPT, 8), RPT)], rndbuf)
    plsc.subcore_barrier()

    def run_spmm(colb, rowb, valb, nch, table):
        def fire(ci):
            p = ci % 2
            return [
                pltpu.async_copy(table.at[colb.at[ci * NGC + i]],
                                 gbufs[p].at[pl.ds(i * G, G)], gsems[p])
                for i in range(NGC)
            ]

        gd = {0: fire(0)}
        sd = {}
        for ci in range(nch):
            p = ci % 2
            if ci + 1 < nch:
                if ci - 1 >= 0:
                    for d in sd[ci - 1]:
                        d.wait()
                gd[ci + 1] = fire(ci + 1)
            for d in gd[ci]:
                d.wait()

            def scale(k16, _, ci=ci, p=p):
                vv = valb[pl.ds(ci * CHUNK + k16 * L, L)]
                for j in range(L):
                    v = vv[j]
                    row = k16 * L + j
                    for q in range(DF // L):
                        sl = pl.ds(q * L, L)
                        gbufs[p][row, sl] = gbufs[p][row, sl] * v
                return 0

            lax.fori_loop(0, CHUNK // L, scale, 0)
            sd[ci] = [
                pltpu.async_copy(gbufs[p].at[pl.ds(i * G, G)],
                                 acc.at[rowb.at[ci * NGC + i]], ssem, add=True)
                for i in range(NGC)
            ]
        for ci in range(max(0, nch - 2), nch):
            for d in sd[ci]:
                d.wait()

    run_spmm(dcol, drow, dval, PTD // CHUNK, x)
    plsc.subcore_barrier()

    sl = pl.ds(s * RPT, RPT)
    pltpu.sync_copy(acc.at[sl], gb0.at[pl.ds(0, RPT)])

    def rscale(k16, _):
        vv = rndbuf[pl.ds(k16 * L, L)]
        for j in range(L):
            v = vv[j]
            row = k16 * L + j
            for q in range(DF // L):
                qsl = pl.ds(q * L, L)
                gb0[row, qsl] = gb0[row, qsl] * v
        return 0

    lax.fori_loop(0, RPT // L, rscale, 0)
    pltpu.sync_copy(gb0.at[pl.ds(0, RPT)], t_stage.at[pl.ds(c * N + s * RPT, RPT)])

    cN = c * N

    def bump(g, _):
        for j in range(G // L):
            jsl = pl.ds(j * L, L)
            dcol[g, jsl] = dcol[g, jsl] + cN
        return 0

    lax.fori_loop(0, DGT, bump, 0)

    _zero_rows(gb0, RPT)
    pltpu.sync_copy(gb0.at[pl.ds(0, RPT)], acc.at[sl])
    plsc.subcore_barrier()

    run_spmm(dcol, drow, dval, PTD // CHUNK, t_stage)
    plsc.subcore_barrier()

    HPC = N // NC // NS
    s2sl = pl.ds(c * (N // NC) + s * HPC, HPC)
    pltpu.sync_copy(acc.at[s2sl], s2_out.at[s2sl])
    plsc.subcore_barrier()

    _zero_rows(gb0, RPT)
    pltpu.sync_copy(gb0.at[pl.ds(0, RPT)], acc.at[sl])
    plsc.subcore_barrier()

    run_spmm(acol, arow, aval, PTA // CHUNK, x)
    plsc.subcore_barrier()
    pltpu.sync_copy(acc.at[sl], ap_out.at[c, sl])


_mega = pl.kernel(
    _mega_body,
    out_type=(
        jax.ShapeDtypeStruct((NC * N, DF), jnp.float32),
        jax.ShapeDtypeStruct((N, DF), jnp.float32),
        jax.ShapeDtypeStruct((NC, N, DF), jnp.float32),
    ),
    mesh=_mesh,
    scratch_types=[
        pltpu.VMEM_SHARED((N, DF), jnp.float32),
        pltpu.VMEM((DGT, G), jnp.int32),
        pltpu.VMEM((DGT, G), jnp.int32),
        pltpu.VMEM((PTD,), jnp.float32),
        pltpu.VMEM((AGT, G), jnp.int32),
        pltpu.VMEM((AGT, G), jnp.int32),
        pltpu.VMEM((PTA,), jnp.float32),
        pltpu.VMEM((RPT,), jnp.float32),
        pltpu.VMEM((CHUNK, DF), jnp.float32),
        pltpu.VMEM((CHUNK, DF), jnp.float32),
        pltpu.SemaphoreType.DMA,
        pltpu.SemaphoreType.DMA,
        pltpu.SemaphoreType.DMA,
    ],
)

_BLK = 512


def _combine_body(coef_ref, s2_ref, a_ref, h0_ref, w_ref, out_ref):
    th = coef_ref[0, 0]
    c1 = coef_ref[0, 1]
    c2 = coef_ref[0, 2]
    c3 = coef_ref[0, 3]
    sup = (c1 * s2_ref[...]
           + c2 * (a_ref[0] + a_ref[1])
           + c3 * h0_ref[...])
    out_ref[...] = th * jnp.dot(
        sup, w_ref[...], preferred_element_type=jnp.float32
    ) + (1.0 - th) * sup


def _combine(coefs, s2, ap, h0, w):
    return pl.pallas_call(
        _combine_body,
        grid=(N // _BLK,),
        in_specs=[
            pl.BlockSpec(memory_space=pltpu.MemorySpace.SMEM),
            pl.BlockSpec((_BLK, DF), lambda i: (i, 0)),
            pl.BlockSpec((NC, _BLK, DF), lambda i: (0, i, 0)),
            pl.BlockSpec((_BLK, DF), lambda i: (i, 0)),
            pl.BlockSpec((DF, DF), lambda i: (0, 0)),
        ],
        out_specs=pl.BlockSpec((_BLK, DF), lambda i: (i, 0)),
        out_shape=jax.ShapeDtypeStruct((N, DF), jnp.float32),
    )(coefs, s2, ap, h0, w)


def kernel(input, h0, adj_rows, adj_cols, adj_vals, d_rows, d_cols, d_vals,
           lamda, alpha, l, gamma, weight):
    x = input
    d1r = d_rows[1].reshape(-1, G)
    d1c = d_cols[1].reshape(-1, G)
    d1v = d_vals[1]
    rnd1 = jax.random.uniform(jax.random.key(42), (2 * N, 1), dtype=jnp.float32)[N:, 0]

    _t, s2, ap = _mega(d1r, d1c, d1v,
                       adj_rows.reshape(-1, G), adj_cols.reshape(-1, G), adj_vals,
                       rnd1, x)

    theta = jnp.log(lamda / l + 1.0)
    af = jnp.float32(alpha)
    gf = jnp.float32(gamma)
    coefs = jnp.stack(
        [jnp.float32(theta), (1.0 - af) * gf, (1.0 - af) * (1.0 - gf), af]
    ).reshape(1, 4)

    return _combine(coefs, s2, ap, h0, weight)


# speedup vs baseline: 14.3516x; 1.1460x over previous
import jax
import jax.numpy as jnp
from jax import lax
from jax.experimental import pallas as pl
from jax.experimental.pallas import tpu as pltpu
from jax.experimental.pallas import tpu_sc as plsc

N = 4096
DF = 128
NNZ_ADJ = 131072
NNZ_D = 16384
NC = 2
NS = 16
NW = NC * NS
L = 16
G = 128
RPT = N // NS

PP = 4
PTD = NNZ_D // NS
PTA = NNZ_ADJ // NW
DGT = PTD // G
AGT = PTA // G

_mesh = plsc.VectorSubcoreMesh(
    core_axis_name="c", subcore_axis_name="s", num_cores=NC, num_subcores=NS
)


def _mega_body(d1r, d1c, d1v, a_r, a_c, a_v, rnd, x, zeros,
               t_stage, s2_out, ap_out,
               acc, dcol, drow, dval, acol, arow, aval, rndbuf,
               gb0, gb1, gb2, gb3,
               gsem0, gsem1, gsem2, gsem3,
               ssem0, ssem1, ssem2, ssem3, psem):
    c = lax.axis_index("c")
    s = lax.axis_index("s")
    wid = s * NC + c
    gbufs = (gb0, gb1, gb2, gb3)
    gsems = (gsem0, gsem1, gsem2, gsem3)
    ssems = (ssem0, ssem1, ssem2, ssem3)
    sl = pl.ds(s * RPT, RPT)

    pre = [
        pltpu.async_copy(zeros.at[sl], acc.at[sl], psem),
        pltpu.async_copy(d1c.at[pl.ds(s * DGT, DGT)], dcol, psem),
        pltpu.async_copy(d1r.at[pl.ds(s * DGT, DGT)], drow, psem),
        pltpu.async_copy(d1v.at[pl.ds(pl.multiple_of(s * PTD, 8), PTD)], dval, psem),
        pltpu.async_copy(a_c.at[pl.ds(wid * AGT, AGT)], acol, psem),
        pltpu.async_copy(a_r.at[pl.ds(wid * AGT, AGT)], arow, psem),
        pltpu.async_copy(a_v.at[pl.ds(pl.multiple_of(wid * PTA, 8), PTA)], aval, psem),
        pltpu.async_copy(rnd.at[pl.ds(pl.multiple_of(s * RPT, 8), RPT)], rndbuf, psem),
    ]
    for d in pre:
        d.wait()
    plsc.subcore_barrier()

    def run_spmm(colb, rowb, valb, nch, table):
        assert nch % PP == 0 and nch >= PP

        def fire_gather(idx, p):
            pltpu.async_copy(table.at[colb.at[idx]], gbufs[p], gsems[p])

        def gwait(p):
            pltpu.make_async_copy(table.at[colb.at[0]], gbufs[p], gsems[p]).wait()

        def fire_scatter(idx, p):
            pltpu.async_copy(gbufs[p], acc.at[rowb.at[idx]], ssems[p], add=True)

        def swait(p):
            pltpu.make_async_copy(gbufs[p], acc.at[rowb.at[0]], ssems[p]).wait()

        def scale(ci, p):
            def body(k16, _):
                vv = valb[pl.ds(ci * G + k16 * L, L)]
                for j in range(L):
                    v = vv[j]
                    row = k16 * L + j
                    for q in range(DF // L):
                        qsl = pl.ds(q * L, L)
                        gbufs[p][row, qsl] = gbufs[p][row, qsl] * v
                return 0

            lax.fori_loop(0, G // L, body, 0)

        fire_gather(0, 0)
        fire_gather(1, 1)

        def group(g, _):
            ci = g * PP
            for pos in range(PP):
                cur = ci + pos
                nxt = cur + 2
                pn = (pos + 2) % PP

                @pl.when(jnp.logical_and(nxt >= PP, nxt < nch))
                def _():
                    swait(pn)

                @pl.when(nxt < nch)
                def _():
                    fire_gather(nxt, pn)

                gwait(pos)
                scale(cur, pos)
                fire_scatter(cur, pos)
            return 0

        lax.fori_loop(0, nch // PP, group, 0)
        for p in range(PP):
            swait(p)

    run_spmm(dcol, drow, dval, DGT, x)
    plsc.subcore_barrier()

    for h in range(RPT // G):
        hsl = pl.ds(s * RPT + h * G, G)
        pltpu.sync_copy(acc.at[hsl], gb0)

        def rscale(k16, _, h=h):
            vv = rndbuf[pl.ds(h * G + k16 * L, L)]
            for j in range(L):
                v = vv[j]
                row = k16 * L + j
                for q in range(DF // L):
                    qsl = pl.ds(q * L, L)
                    gb0[row, qsl] = gb0[row, qsl] * v
            return 0

        lax.fori_loop(0, G // L, rscale, 0)
        pltpu.sync_copy(gb0, t_stage.at[pl.ds(c * N + s * RPT + h * G, G)])

    cN = c * N

    def bump(g, _):
        for j in range(G // L):
            jsl = pl.ds(j * L, L)
            dcol[g, jsl] = dcol[g, jsl] + cN
        return 0

    lax.fori_loop(0, DGT, bump, 0)
    pltpu.sync_copy(zeros.at[sl], acc.at[sl])
    plsc.subcore_barrier()

    run_spmm(dcol, drow, dval, DGT, t_stage)
    plsc.subcore_barrier()

    @pl.when(c == 0)
    def _():
        pltpu.sync_copy(acc.at[sl], s2_out.at[sl])

    pltpu.sync_copy(zeros.at[sl], acc.at[sl])
    plsc.subcore_barrier()

    run_spmm(acol, arow, aval, AGT, x)
    plsc.subcore_barrier()
    pltpu.sync_copy(acc.at[sl], ap_out.at[c, sl])


_mega = pl.kernel(
    _mega_body,
    out_type=(
        jax.ShapeDtypeStruct((NC * N, DF), jnp.float32),
        jax.ShapeDtypeStruct((N, DF), jnp.float32),
        jax.ShapeDtypeStruct((NC, N, DF), jnp.float32),
    ),
    mesh=_mesh,
    scratch_types=[
        pltpu.VMEM_SHARED((N, DF), jnp.float32),
        pltpu.VMEM((DGT, G), jnp.int32),
        pltpu.VMEM((DGT, G), jnp.int32),
        pltpu.VMEM((PTD,), jnp.float32),
        pltpu.VMEM((AGT, G), jnp.int32),
        pltpu.VMEM((AGT, G), jnp.int32),
        pltpu.VMEM((PTA,), jnp.float32),
        pltpu.VMEM((RPT,), jnp.float32),
        pltpu.VMEM((G, DF), jnp.float32),
        pltpu.VMEM((G, DF), jnp.float32),
        pltpu.VMEM((G, DF), jnp.float32),
        pltpu.VMEM((G, DF), jnp.float32),
        pltpu.SemaphoreType.DMA,
        pltpu.SemaphoreType.DMA,
        pltpu.SemaphoreType.DMA,
        pltpu.SemaphoreType.DMA,
        pltpu.SemaphoreType.DMA,
        pltpu.SemaphoreType.DMA,
        pltpu.SemaphoreType.DMA,
        pltpu.SemaphoreType.DMA,
        pltpu.SemaphoreType.DMA,
    ],
)

_BLK = 512


def _combine_body(coef_ref, s2_ref, a_ref, h0_ref, w_ref, out_ref):
    th = coef_ref[0, 0]
    c1 = coef_ref[0, 1]
    c2 = coef_ref[0, 2]
    c3 = coef_ref[0, 3]
    sup = (c1 * s2_ref[...]
           + c2 * (a_ref[0] + a_ref[1])
           + c3 * h0_ref[...])
    out_ref[...] = th * jnp.dot(
        sup, w_ref[...], preferred_element_type=jnp.float32
    ) + (1.0 - th) * sup


def _combine(coefs, s2, ap, h0, w):
    return pl.pallas_call(
        _combine_body,
        grid=(N // _BLK,),
        in_specs=[
            pl.BlockSpec(memory_space=pltpu.MemorySpace.SMEM),
            pl.BlockSpec((_BLK, DF), lambda i: (i, 0)),
            pl.BlockSpec((NC, _BLK, DF), lambda i: (0, i, 0)),
            pl.BlockSpec((_BLK, DF), lambda i: (i, 0)),
            pl.BlockSpec((DF, DF), lambda i: (0, 0)),
        ],
        out_specs=pl.BlockSpec((_BLK, DF), lambda i: (i, 0)),
        out_shape=jax.ShapeDtypeStruct((N, DF), jnp.float32),
    )(coefs, s2, ap, h0, w)


def kernel(input, h0, adj_rows, adj_cols, adj_vals, d_rows, d_cols, d_vals,
           lamda, alpha, l, gamma, weight):
    x = input
    d1r = d_rows[1].reshape(-1, G)
    d1c = d_cols[1].reshape(-1, G)
    d1v = d_vals[1]
    rnd1 = jax.random.uniform(jax.random.key(42), (2 * N, 1), dtype=jnp.float32)[N:, 0]

    zeros = jnp.zeros((N, DF), jnp.float32)
    _t, s2, ap = _mega(d1r, d1c, d1v,
                       adj_rows.reshape(-1, G), adj_cols.reshape(-1, G), adj_vals,
                       rnd1, x, zeros)

    theta = jnp.log(lamda / l + 1.0)
    af = jnp.float32(alpha)
    gf = jnp.float32(gamma)
    coefs = jnp.stack(
        [jnp.float32(theta), (1.0 - af) * gf, (1.0 - af) * (1.0 - gf), af]
    ).reshape(1, 4)

    return _combine(coefs, s2, ap, h0, weight)
